# Initial kernel scaffold; baseline (speedup 1.0000x reference)
import jax
import jax.numpy as jnp
from jax.experimental import pallas as pl


def kernel(x, edge_index, edge_weight, enc_W1, enc_b1, enc_W2, enc_b2, gcn_W0, gcn_b0, gcn_W1, gcn_b1, gcn_W2, gcn_b2, dec_W1, dec_b1, dec_W2, dec_b2):
    raise NotImplementedError("write your pallas kernel here")


# speedup vs baseline: 2.2811x; 2.2811x over previous
import functools

import jax
import jax.numpy as jnp
from jax.experimental import pallas as pl

N = 10000
D = 128
ROW_BLOCK = 1000


def _mm_kernel(h_ref, w_ref, b_ref, o_ref, *, relu):
    acc = jnp.dot(h_ref[...], w_ref[...],
                  preferred_element_type=jnp.float32,
                  precision=jax.lax.Precision.HIGHEST)
    acc = acc + b_ref[...]
    if relu:
        acc = jnp.maximum(acc, 0.0)
    o_ref[...] = acc


def _mm(h, w, b, relu):
    grid = (h.shape[0] // ROW_BLOCK,)
    return pl.pallas_call(
        functools.partial(_mm_kernel, relu=relu),
        grid=grid,
        in_specs=[
            pl.BlockSpec((ROW_BLOCK, D), lambda i: (i, 0)),
            pl.BlockSpec((D, D), lambda i: (0, 0)),
            pl.BlockSpec((D,), lambda i: (0,)),
        ],
        out_specs=pl.BlockSpec((ROW_BLOCK, D), lambda i: (i, 0)),
        out_shape=jax.ShapeDtypeStruct((h.shape[0], D), jnp.float32),
    )(h, w, b)


def kernel(x, edge_index, edge_weight,
           enc_W1, enc_b1, enc_W2, enc_b2,
           gcn_W0, gcn_b0, gcn_W1, gcn_b1, gcn_W2, gcn_b2,
           dec_W1, dec_b1, dec_W2, dec_b2):
    n = x.shape[0]
    row = edge_index[0].astype(jnp.int32)
    col = edge_index[1].astype(jnp.int32)
    ew = edge_weight

    deg = jnp.zeros((n,), jnp.float32).at[col].add(ew) + 1.0
    dis = deg ** -0.5

    h = _mm(_mm(x, enc_W1, enc_b1, relu=True), enc_W2, enc_b2, relu=False)
    for W, b in ((gcn_W0, gcn_b0), (gcn_W1, gcn_b1), (gcn_W2, gcn_b2)):
        y = dis[:, None] * _mm(h, W, jnp.zeros((D,), jnp.float32), relu=False)
        acc = jnp.zeros_like(y).at[col].add(ew[:, None] * y[row])
        h = jnp.maximum(dis[:, None] * (acc + y) + b, 0.0)
    out = _mm(_mm(h, dec_W1, dec_b1, relu=True), dec_W2, dec_b2, relu=False)
    return out
